# baseline (speedup 1.0000x reference)
import functools

import jax
import jax.numpy as jnp
from jax import lax
from jax.experimental import pallas as pl
from jax.experimental.pallas import tpu as pltpu
from jax.experimental.pallas import tpu_sc as plsc

_B, _S, _D = 16, 2048, 768
_W = 1024
_NL = 9
_PAD = 16
_ROWS = _B * _S
_TBLK = 512

_NC, _NS = 2, 16


def _tc1_body(x_ref, w_ref, p_ref):
    x = x_ref[0]
    w = w_ref[...]
    logits = jnp.dot(x, w, preferred_element_type=jnp.float32)
    lane = lax.broadcasted_iota(jnp.int32, logits.shape, 1)
    ml = jnp.where(lane < _NL, logits, -1e30)
    m = jnp.max(ml, axis=-1, keepdims=True)
    e = jnp.exp(ml - m)
    p = e / jnp.sum(e, axis=-1, keepdims=True)
    p = jnp.where(lane == _NL, 1.0, p)
    p_ref[0] = p.T


def _sc_seg_sum(probs_t, word_ids):
    mesh = plsc.VectorSubcoreMesh(core_axis_name="c", subcore_axis_name="s")
    rows_per_core = _B // _NC

    @functools.partial(
        pl.kernel,
        mesh=mesh,
        out_type=jax.ShapeDtypeStruct((_B, _PAD * _W), jnp.float32),
        compiler_params=pltpu.CompilerParams(needs_layout_passes=False),
        scratch_types=[
            pltpu.VMEM((_S,), jnp.int32),
            pltpu.VMEM((_PAD, _S), jnp.float32),
            pltpu.VMEM((_PAD * _W,), jnp.float32),
        ],
    )
    def k(probs_hbm, wid_hbm, out_hbm, idx_v, pv, acc):
        c = lax.axis_index("c")
        s = lax.axis_index("s")
        b = c * rows_per_core + s

        @pl.when(s < rows_per_core)
        def _():
            pltpu.sync_copy(wid_hbm.at[b], idx_v)
            pltpu.sync_copy(probs_hbm.at[b], pv)

            zeros = jnp.zeros((16,), jnp.float32)

            def zero_body(t, carry):
                acc[pl.ds(t * 16, 16)] = zeros
                return carry

            lax.fori_loop(0, _PAD * _W // 16, zero_body, 0)

            def body(t, carry):
                sl = pl.ds(t * 16, 16)
                idx = idx_v[sl]
                for j in range(_NL + 1):
                    plsc.addupdate_scatter(acc, [idx + (j * _W)], pv[j, sl])
                return carry

            lax.fori_loop(0, _S // 16, body, 0)
            pltpu.sync_copy(acc, out_hbm.at[b])

    return k(probs_t, word_ids)


def _tc2_body(s_ref, lab_ref, avg_ref, loss_ref):
    data = s_ref[...]
    cnt = data[:, _NL:_NL + 1, :]
    avg = data / jnp.maximum(cnt, 1.0)
    row = lax.broadcasted_iota(jnp.int32, avg.shape, 1)
    ml = jnp.where(row < _NL, avg, -1e30)
    m = jnp.max(ml, axis=1, keepdims=True)
    se = jnp.sum(jnp.exp(ml - m), axis=1, keepdims=True)
    lab = lab_ref[...]
    picked = jnp.sum(jnp.where(row == lab, avg, 0.0), axis=1, keepdims=True)
    nll = m + jnp.log(se) - picked
    avg_ref[...] = jnp.swapaxes(avg[:, :_NL, :], 1, 2)
    loss_ref[0, 0] = jnp.sum(nll) * (1.0 / (_B * _W))


def kernel(sent_logits, word_ids, labels, W_mlp):
    w16 = jnp.zeros((_D, _PAD), jnp.float32).at[:, :_NL].set(W_mlp)

    probs_t = pl.pallas_call(
        _tc1_body,
        grid=(_B, _S // _TBLK),
        in_specs=[
            pl.BlockSpec((1, _TBLK, _D), lambda i, j: (i, j, 0)),
            pl.BlockSpec((_D, _PAD), lambda i, j: (0, 0)),
        ],
        out_specs=pl.BlockSpec((1, _PAD, _TBLK), lambda i, j: (i, 0, j)),
        out_shape=jax.ShapeDtypeStruct((_B, _PAD, _S), jnp.float32),
    )(sent_logits, w16)

    sums = _sc_seg_sum(probs_t, word_ids).reshape(_B, _PAD, _W)

    avg, loss = pl.pallas_call(
        _tc2_body,
        in_specs=[
            pl.BlockSpec((_B, _PAD, _W), lambda: (0, 0, 0)),
            pl.BlockSpec((_B, 1, _W), lambda: (0, 0, 0)),
        ],
        out_specs=[
            pl.BlockSpec((_B, _W, _NL), lambda: (0, 0, 0)),
            pl.BlockSpec((1, 1), lambda: (0, 0), memory_space=pltpu.SMEM),
        ],
        out_shape=[
            jax.ShapeDtypeStruct((_B, _W, _NL), jnp.float32),
            jax.ShapeDtypeStruct((1, 1), jnp.float32),
        ],
    )(sums, labels.reshape(_B, 1, _W))

    return avg, loss[0, 0]


# speedup vs baseline: 1.3305x; 1.3305x over previous
import functools

import jax
import jax.numpy as jnp
from jax import lax
from jax.experimental import pallas as pl
from jax.experimental.pallas import tpu as pltpu
from jax.experimental.pallas import tpu_sc as plsc

_B, _S, _D = 16, 2048, 768
_W = 1024
_NL = 9
_PAD = 16
_ROWS = _B * _S
_TBLK = 2048

_NC, _NS = 2, 16


def _tc1_body(x_ref, w_ref, p_ref):
    x = x_ref[0]
    w = w_ref[...]
    logits = jnp.dot(x, w, preferred_element_type=jnp.float32)
    lane = lax.broadcasted_iota(jnp.int32, logits.shape, 1)
    ml = jnp.where(lane < _NL, logits, -1e30)
    m = jnp.max(ml, axis=-1, keepdims=True)
    e = jnp.exp(ml - m)
    p = e / jnp.sum(e, axis=-1, keepdims=True)
    p = jnp.where(lane == _NL, 1.0, p)
    p_ref[0] = p.T


def _sc_seg_sum(probs_t, word_ids):
    mesh = plsc.VectorSubcoreMesh(core_axis_name="c", subcore_axis_name="s")
    rows_per_core = _B // _NC

    @functools.partial(
        pl.kernel,
        mesh=mesh,
        out_type=jax.ShapeDtypeStruct((_B, _PAD * _W), jnp.float32),
        compiler_params=pltpu.CompilerParams(needs_layout_passes=False),
        scratch_types=[
            pltpu.VMEM((_S,), jnp.int32),
            pltpu.VMEM((_PAD, _S), jnp.float32),
            pltpu.VMEM((_PAD * _W,), jnp.float32),
        ],
    )
    def k(probs_hbm, wid_hbm, out_hbm, idx_v, pv, acc):
        c = lax.axis_index("c")
        s = lax.axis_index("s")
        b = c * rows_per_core + s

        @pl.when(s < rows_per_core)
        def _():
            pltpu.sync_copy(wid_hbm.at[b], idx_v)
            pltpu.sync_copy(probs_hbm.at[b], pv)

            zeros = jnp.zeros((16,), jnp.float32)

            def zero_body(t, carry):
                acc[pl.ds(t * 16, 16)] = zeros
                return carry

            lax.fori_loop(0, _PAD * _W // 16, zero_body, 0)

            def body(t, carry):
                sl = pl.ds(t * 16, 16)
                idx = idx_v[sl]
                for j in range(_NL + 1):
                    plsc.addupdate_scatter(acc, [idx + (j * _W)], pv[j, sl])
                return carry

            lax.fori_loop(0, _S // 16, body, 0)
            pltpu.sync_copy(acc, out_hbm.at[b])

    return k(probs_t, word_ids)


def _tc2_body(s_ref, lab_ref, avg_ref, loss_ref):
    data = s_ref[...]
    cnt = data[:, _NL:_NL + 1, :]
    avg = data / jnp.maximum(cnt, 1.0)
    row = lax.broadcasted_iota(jnp.int32, avg.shape, 1)
    ml = jnp.where(row < _NL, avg, -1e30)
    m = jnp.max(ml, axis=1, keepdims=True)
    se = jnp.sum(jnp.exp(ml - m), axis=1, keepdims=True)
    lab = lab_ref[...]
    picked = jnp.sum(jnp.where(row == lab, avg, 0.0), axis=1, keepdims=True)
    nll = m + jnp.log(se) - picked
    avg_ref[...] = jnp.swapaxes(avg[:, :_NL, :], 1, 2)
    loss_ref[0, 0] = jnp.sum(nll) * (1.0 / (_B * _W))


def kernel(sent_logits, word_ids, labels, W_mlp):
    w16 = jnp.zeros((_D, _PAD), jnp.float32).at[:, :_NL].set(W_mlp)

    probs_t = pl.pallas_call(
        _tc1_body,
        grid=(_B, _S // _TBLK),
        in_specs=[
            pl.BlockSpec((1, _TBLK, _D), lambda i, j: (i, j, 0)),
            pl.BlockSpec((_D, _PAD), lambda i, j: (0, 0)),
        ],
        out_specs=pl.BlockSpec((1, _PAD, _TBLK), lambda i, j: (i, 0, j)),
        out_shape=jax.ShapeDtypeStruct((_B, _PAD, _S), jnp.float32),
    )(sent_logits, w16)

    sums = _sc_seg_sum(probs_t, word_ids).reshape(_B, _PAD, _W)

    avg, loss = pl.pallas_call(
        _tc2_body,
        in_specs=[
            pl.BlockSpec((_B, _PAD, _W), lambda: (0, 0, 0)),
            pl.BlockSpec((_B, 1, _W), lambda: (0, 0, 0)),
        ],
        out_specs=[
            pl.BlockSpec((_B, _W, _NL), lambda: (0, 0, 0)),
            pl.BlockSpec((1, 1), lambda: (0, 0), memory_space=pltpu.SMEM),
        ],
        out_shape=[
            jax.ShapeDtypeStruct((_B, _W, _NL), jnp.float32),
            jax.ShapeDtypeStruct((1, 1), jnp.float32),
        ],
    )(sums, labels.reshape(_B, 1, _W))

    return avg, loss[0, 0]


# speedup vs baseline: 1.3928x; 1.0469x over previous
import functools

import jax
import jax.numpy as jnp
from jax import lax
from jax.experimental import pallas as pl
from jax.experimental.pallas import tpu as pltpu
from jax.experimental.pallas import tpu_sc as plsc

_B, _S, _D = 16, 2048, 768
_W = 1024
_NL = 9
_PAD = 16
_ROWS = _B * _S
_TBLK = 2048

_NC, _NS = 2, 16


def _tc1_body(x_ref, w_ref, p_ref):
    x = x_ref[0]
    w = w_ref[...]
    logits = jnp.dot(x, w, preferred_element_type=jnp.float32)
    lane = lax.broadcasted_iota(jnp.int32, logits.shape, 1)
    ml = jnp.where(lane < _NL, logits, -1e30)
    m = jnp.max(ml, axis=-1, keepdims=True)
    e = jnp.exp(ml - m)
    p = e / jnp.sum(e, axis=-1, keepdims=True)
    p = jnp.where(lane == _NL, 1.0, p)
    p_ref[0] = p.T


def _sc_seg_sum(probs_t, word_ids):
    mesh = plsc.VectorSubcoreMesh(core_axis_name="c", subcore_axis_name="s")
    rows_per_core = _B // _NC
    half_s = _S // 2

    @functools.partial(
        pl.kernel,
        mesh=mesh,
        out_type=jax.ShapeDtypeStruct((2 * _B, _PAD * _W), jnp.float32),
        compiler_params=pltpu.CompilerParams(needs_layout_passes=False),
        scratch_types=[
            pltpu.VMEM((half_s,), jnp.int32),
            pltpu.VMEM((_PAD, half_s), jnp.float32),
            pltpu.VMEM((_PAD * _W,), jnp.float32),
        ],
    )
    def k(probs_hbm, wid_hbm, out_hbm, idx_v, pv, acc):
        c = lax.axis_index("c")
        s = lax.axis_index("s")
        b = c * rows_per_core + s // 2
        half = s % 2

        pltpu.sync_copy(wid_hbm.at[b, pl.ds(half * half_s, half_s)], idx_v)
        pltpu.sync_copy(probs_hbm.at[b, :, pl.ds(half * half_s, half_s)], pv)

        zeros = jnp.zeros((16,), jnp.float32)

        def zero_body(t, carry):
            acc[pl.ds(t * 16, 16)] = zeros
            return carry

        lax.fori_loop(0, _PAD * _W // 16, zero_body, 0)

        def body(t, carry):
            sl = pl.ds(t * 16, 16)
            idx = idx_v[sl]
            for j in range(_NL + 1):
                plsc.addupdate_scatter(acc, [idx + (j * _W)], pv[j, sl])
            return carry

        lax.fori_loop(0, half_s // 16, body, 0)
        pltpu.sync_copy(acc, out_hbm.at[2 * b + half])

    return k(probs_t, word_ids)


def _tc2_body(s_ref, lab_ref, avg_ref, loss_ref):
    part = s_ref[...]
    data = part[:, 0] + part[:, 1]
    cnt = data[:, _NL:_NL + 1, :]
    avg = data / jnp.maximum(cnt, 1.0)
    row = lax.broadcasted_iota(jnp.int32, avg.shape, 1)
    ml = jnp.where(row < _NL, avg, -1e30)
    m = jnp.max(ml, axis=1, keepdims=True)
    se = jnp.sum(jnp.exp(ml - m), axis=1, keepdims=True)
    lab = lab_ref[...]
    picked = jnp.sum(jnp.where(row == lab, avg, 0.0), axis=1, keepdims=True)
    nll = m + jnp.log(se) - picked
    avg_ref[...] = jnp.swapaxes(avg[:, :_NL, :], 1, 2)
    loss_ref[0, 0] = jnp.sum(nll) * (1.0 / (_B * _W))


def kernel(sent_logits, word_ids, labels, W_mlp):
    w16 = jnp.zeros((_D, _PAD), jnp.float32).at[:, :_NL].set(W_mlp)

    probs_t = pl.pallas_call(
        _tc1_body,
        grid=(_B, _S // _TBLK),
        in_specs=[
            pl.BlockSpec((1, _TBLK, _D), lambda i, j: (i, j, 0)),
            pl.BlockSpec((_D, _PAD), lambda i, j: (0, 0)),
        ],
        out_specs=pl.BlockSpec((1, _PAD, _TBLK), lambda i, j: (i, 0, j)),
        out_shape=jax.ShapeDtypeStruct((_B, _PAD, _S), jnp.float32),
    )(sent_logits, w16)

    sums = _sc_seg_sum(probs_t, word_ids).reshape(_B, 2, _PAD, _W)

    avg, loss = pl.pallas_call(
        _tc2_body,
        in_specs=[
            pl.BlockSpec((_B, 2, _PAD, _W), lambda: (0, 0, 0, 0)),
            pl.BlockSpec((_B, 1, _W), lambda: (0, 0, 0)),
        ],
        out_specs=[
            pl.BlockSpec((_B, _W, _NL), lambda: (0, 0, 0)),
            pl.BlockSpec((1, 1), lambda: (0, 0), memory_space=pltpu.SMEM),
        ],
        out_shape=[
            jax.ShapeDtypeStruct((_B, _W, _NL), jnp.float32),
            jax.ShapeDtypeStruct((1, 1), jnp.float32),
        ],
    )(sums, labels.reshape(_B, 1, _W))

    return avg, loss[0, 0]
